# baseline (speedup 1.0000x reference)
import jax
import jax.numpy as jnp
from jax import lax
from jax.experimental import pallas as pl
from jax.experimental.pallas import tpu as pltpu
from jax.experimental.pallas import tpu_sc as plsc

_VOCAB2 = 2048
_DIM = 64
_ROW = 128
_NC = 2
_NS = 16
_LANES = 16
_PCHUNK = 80


def _tanh16(x):
    e = jnp.exp(2.0 * x)
    return 1.0 - 2.0 / (e + 1.0)


def _body(ue_hbm, uo_hbm, s_hbm, out_hbm, shared, ttile, uev, uov, buf, sem):
    c = lax.axis_index("c")
    s = lax.axis_index("s")
    wid = c * _NS + s
    rows_per_tile = _VOCAB2 // _NS
    n_chunks = uev.shape[0]

    pltpu.sync_copy(s_hbm.at[pl.ds(s * rows_per_tile, rows_per_tile)], ttile)

    def tanh_row(r, _):
        for c8 in range(_ROW // _LANES):
            sl = pl.ds(c8 * _LANES, _LANES)
            ttile[r, sl] = _tanh16(ttile[r, sl])
        return _

    lax.fori_loop(0, rows_per_tile, tanh_row, None)
    pltpu.sync_copy(ttile, shared.at[pl.ds(s * rows_per_tile, rows_per_tile)])
    plsc.subcore_barrier()

    pltpu.sync_copy(ue_hbm.at[pl.ds(wid * n_chunks, n_chunks)], uev)
    pltpu.sync_copy(uo_hbm.at[pl.ds(wid * n_chunks, n_chunks)], uov)
    out_base = wid * n_chunks * _PCHUNK

    def chunk(i, _):
        pltpu.async_copy(shared.at[uev.at[i]], buf, sem).wait()
        pltpu.async_copy(shared.at[uov.at[i]], buf, sem, add=True).wait()
        pltpu.sync_copy(buf, out_hbm.at[pl.ds(out_base + i * _PCHUNK, _PCHUNK)])
        return _

    lax.fori_loop(0, n_chunks, chunk, None)


def kernel(input, table):
    b, l = input.shape
    n = b * l
    npair = n // 2
    nw = _NC * _NS
    chunks_per_tile = npair // (nw * _PCHUNK)
    v, d = table.shape

    flat = input.reshape(-1).astype(jnp.int32)
    u = 2 * flat + (jnp.arange(n, dtype=jnp.int32) % 2)
    ue = u[0::2].reshape(npair // _PCHUNK, _PCHUNK)
    uo = u[1::2].reshape(npair // _PCHUNK, _PCHUNK)

    tpad = jnp.zeros((_VOCAB2 // 2, _DIM), table.dtype).at[:v].set(table)
    spre = jnp.zeros((_VOCAB2, _ROW), jnp.float32)
    spre = spre.at[0::2, :_DIM].set(tpad)
    spre = spre.at[1::2, _DIM:].set(tpad)

    mesh = plsc.VectorSubcoreMesh(core_axis_name="c", subcore_axis_name="s")
    run = pl.kernel(
        _body,
        out_type=jax.ShapeDtypeStruct((npair, _ROW), jnp.float32),
        mesh=mesh,
        scratch_types=[
            pltpu.VMEM_SHARED((_VOCAB2, _ROW), jnp.float32),
            pltpu.VMEM((_VOCAB2 // _NS, _ROW), jnp.float32),
            pltpu.VMEM((chunks_per_tile, _PCHUNK), jnp.int32),
            pltpu.VMEM((chunks_per_tile, _PCHUNK), jnp.int32),
            pltpu.VMEM((_PCHUNK, _ROW), jnp.float32),
            pltpu.SemaphoreType.DMA,
        ],
    )
    out = run(ue, uo, spre)
    return out.reshape(b, l, _DIM)


# speedup vs baseline: 35.7293x; 35.7293x over previous
import jax
import jax.numpy as jnp
from jax import lax
from jax.experimental import pallas as pl
from jax.experimental.pallas import tpu as pltpu
from jax.experimental.pallas import tpu_sc as plsc

_VOCAB2 = 2048
_DIM = 64
_ROW = 128
_NC = 2
_NS = 16
_LANES = 16
_PCHUNK = 80


def _tanh16(x):
    e = jnp.exp(2.0 * x)
    return 1.0 - 2.0 / (e + 1.0)


def _body(ue_hbm, uo_hbm, s_hbm, out_hbm, shared, ttile, uev, uov, buf, sem):
    c = lax.axis_index("c")
    s = lax.axis_index("s")
    wid = c * _NS + s
    rows_per_tile = _VOCAB2 // _NS
    n_chunks = uev.shape[0]

    pltpu.sync_copy(s_hbm.at[pl.ds(s * rows_per_tile, rows_per_tile)], ttile)

    def tanh_row(r, _):
        for c8 in range(_ROW // _LANES):
            sl = pl.ds(c8 * _LANES, _LANES)
            ttile[r, sl] = _tanh16(ttile[r, sl])
        return _

    lax.fori_loop(0, rows_per_tile, tanh_row, None)
    pltpu.sync_copy(ttile, shared.at[pl.ds(s * rows_per_tile, rows_per_tile)])
    plsc.subcore_barrier()

    pltpu.sync_copy(ue_hbm.at[pl.ds(wid * n_chunks, n_chunks)], uev)
    pltpu.sync_copy(uo_hbm.at[pl.ds(wid * n_chunks, n_chunks)], uov)
    out_base = wid * n_chunks * _PCHUNK

    def chunk(i, _):
        pltpu.async_copy(shared.at[uev.at[i]], buf, sem).wait()
        pltpu.async_copy(shared.at[uov.at[i]], buf, sem, add=True).wait()
        pltpu.sync_copy(buf, out_hbm.at[pl.ds(out_base + i * _PCHUNK, _PCHUNK)])
        return _

    lax.fori_loop(0, n_chunks, chunk, None)


def kernel(input, table):
    b, l = input.shape
    n = b * l
    npair = n // 2
    nw = _NC * _NS
    chunks_per_tile = npair // (nw * _PCHUNK)
    v, d = table.shape

    flat2 = input.reshape(npair, 2).astype(jnp.int32)
    ue = (2 * flat2[:, 0]).reshape(npair // _PCHUNK, _PCHUNK)
    uo = (2 * flat2[:, 1] + 1).reshape(npair // _PCHUNK, _PCHUNK)

    tpad = jnp.zeros((_VOCAB2 // 2, _DIM), jnp.float32).at[:v].set(table)
    spre = jnp.concatenate(
        [tpad, jnp.zeros((_VOCAB2 // 2, _ROW), jnp.float32), tpad], axis=1
    ).reshape(_VOCAB2, _ROW)

    mesh = plsc.VectorSubcoreMesh(core_axis_name="c", subcore_axis_name="s")
    run = pl.kernel(
        _body,
        out_type=jax.ShapeDtypeStruct((npair, _ROW), jnp.float32),
        mesh=mesh,
        scratch_types=[
            pltpu.VMEM_SHARED((_VOCAB2, _ROW), jnp.float32),
            pltpu.VMEM((_VOCAB2 // _NS, _ROW), jnp.float32),
            pltpu.VMEM((chunks_per_tile, _PCHUNK), jnp.int32),
            pltpu.VMEM((chunks_per_tile, _PCHUNK), jnp.int32),
            pltpu.VMEM((_PCHUNK, _ROW), jnp.float32),
            pltpu.SemaphoreType.DMA,
        ],
    )
    out = run(ue, uo, spre)
    return out.reshape(b, l, _DIM)


# speedup vs baseline: 35.9700x; 1.0067x over previous
import jax
import jax.numpy as jnp
from jax import lax
from jax.experimental import pallas as pl
from jax.experimental.pallas import tpu as pltpu
from jax.experimental.pallas import tpu_sc as plsc

_VOCAB_PAD = 1024
_DIM = 64
_ROW = 128
_NC = 2
_NS = 16
_LANES = 16
_PCHUNK = 128


def _tanh16(x):
    e = jnp.exp(2.0 * x)
    return 1.0 - 2.0 / (e + 1.0)


def _body(ue_hbm, uo_hbm, t_hbm, out_hbm, shared, tload, tbuild, uev, uov,
          buf, sem):
    c = lax.axis_index("c")
    s = lax.axis_index("s")
    wid = c * _NS + s
    raw_rows = _VOCAB_PAD // _NS
    pairs = uev.shape[0]
    n_chunks = pairs // _PCHUNK

    pltpu.sync_copy(t_hbm.at[pl.ds(s * raw_rows, raw_rows)], tload)
    zero = jnp.zeros((_LANES,), jnp.float32)

    def build_row(r, _):
        for c4 in range(_DIM // _LANES):
            sl = pl.ds(c4 * _LANES, _LANES)
            sh = pl.ds(_DIM + c4 * _LANES, _LANES)
            x = _tanh16(tload[r, sl])
            tbuild[2 * r, sl] = x
            tbuild[2 * r, sh] = zero
            tbuild[2 * r + 1, sl] = zero
            tbuild[2 * r + 1, sh] = x
        return _

    lax.fori_loop(0, raw_rows, build_row, None)
    pltpu.sync_copy(tbuild,
                    shared.at[pl.ds(s * 2 * raw_rows, 2 * raw_rows)])
    plsc.subcore_barrier()

    pltpu.sync_copy(ue_hbm.at[pl.ds(wid * pairs, pairs)], uev)
    pltpu.sync_copy(uo_hbm.at[pl.ds(wid * pairs, pairs)], uov)
    out_base = wid * pairs

    def chunk(i, _):
        sl = pl.ds(i * _PCHUNK, _PCHUNK)
        pltpu.async_copy(shared.at[uev.at[sl]], buf, sem).wait()
        pltpu.async_copy(shared.at[uov.at[sl]], buf, sem, add=True).wait()
        pltpu.sync_copy(buf, out_hbm.at[pl.ds(out_base + i * _PCHUNK,
                                              _PCHUNK)])
        return _

    lax.fori_loop(0, n_chunks, chunk, None)


def kernel(input, table):
    b, l = input.shape
    n = b * l
    npair = n // 2
    nw = _NC * _NS
    pairs_per_tile = npair // nw
    v, d = table.shape

    flat2 = input.reshape(npair, 2).astype(jnp.int32)
    ue = (2 * flat2[:, 0]).reshape(npair)
    uo = (2 * flat2[:, 1] + 1).reshape(npair)
    tpad = jnp.zeros((_VOCAB_PAD, _DIM), jnp.float32).at[:v].set(table)

    mesh = plsc.VectorSubcoreMesh(core_axis_name="c", subcore_axis_name="s")
    run = pl.kernel(
        _body,
        out_type=jax.ShapeDtypeStruct((npair, _ROW), jnp.float32),
        mesh=mesh,
        scratch_types=[
            pltpu.VMEM_SHARED((2 * _VOCAB_PAD, _ROW), jnp.float32),
            pltpu.VMEM((_VOCAB_PAD // _NS, _DIM), jnp.float32),
            pltpu.VMEM((2 * _VOCAB_PAD // _NS, _ROW), jnp.float32),
            pltpu.VMEM((pairs_per_tile,), jnp.int32),
            pltpu.VMEM((pairs_per_tile,), jnp.int32),
            pltpu.VMEM((_PCHUNK, _ROW), jnp.float32),
            pltpu.SemaphoreType.DMA,
        ],
    )
    out = run(ue, uo, tpad)
    return out.reshape(b, l, _DIM)


# speedup vs baseline: 37.1420x; 1.0326x over previous
import jax
import jax.numpy as jnp
from jax import lax
from jax.experimental import pallas as pl
from jax.experimental.pallas import tpu as pltpu
from jax.experimental.pallas import tpu_sc as plsc

_VOCAB_PAD = 1024
_DIM = 64
_ROW = 128
_NC = 2
_NS = 16
_LANES = 16
_PCHUNK = 128


def _tanh16(x):
    e = jnp.exp(2.0 * x)
    return 1.0 - 2.0 / (e + 1.0)


def _body(ue_hbm, uo_hbm, t_hbm, out_hbm, shared, tload, tbuild, uev, uov,
          buf0, buf1, sem_idx, sem_ge, sem_go, sem_wr):
    c = lax.axis_index("c")
    s = lax.axis_index("s")
    wid = c * _NS + s
    raw_rows = _VOCAB_PAD // _NS
    pairs = uev.shape[0]
    n_chunks = pairs // _PCHUNK

    idx_e = pltpu.async_copy(ue_hbm.at[pl.ds(wid * pairs, pairs)], uev,
                             sem_idx)
    idx_o = pltpu.async_copy(uo_hbm.at[pl.ds(wid * pairs, pairs)], uov,
                             sem_idx)

    pltpu.sync_copy(t_hbm.at[pl.ds(s * raw_rows, raw_rows)], tload)
    zero = jnp.zeros((_LANES,), jnp.float32)

    def build_row(r, _):
        for c4 in range(_DIM // _LANES):
            sl = pl.ds(c4 * _LANES, _LANES)
            sh = pl.ds(_DIM + c4 * _LANES, _LANES)
            x = _tanh16(tload[r, sl])
            tbuild[2 * r, sl] = x
            tbuild[2 * r, sh] = zero
            tbuild[2 * r + 1, sl] = zero
            tbuild[2 * r + 1, sh] = x
        return _

    lax.fori_loop(0, raw_rows, build_row, None)
    pltpu.sync_copy(tbuild,
                    shared.at[pl.ds(s * 2 * raw_rows, 2 * raw_rows)])
    idx_e.wait()
    idx_o.wait()
    plsc.subcore_barrier()

    out_base = wid * pairs
    bufs = (buf0, buf1)

    def gath_e(i):
        return pltpu.async_copy(
            shared.at[uev.at[pl.ds(i * _PCHUNK, _PCHUNK)]],
            bufs[i % 2], sem_ge)

    writes = [None, None]
    ge = gath_e(0)
    for i in range(n_chunks):
        ge.wait()
        go = pltpu.async_copy(
            shared.at[uov.at[pl.ds(i * _PCHUNK, _PCHUNK)]],
            bufs[i % 2], sem_go, add=True)
        if i + 1 < n_chunks:
            if writes[(i + 1) % 2] is not None:
                writes[(i + 1) % 2].wait()
            ge = gath_e(i + 1)
        go.wait()
        writes[i % 2] = pltpu.async_copy(
            bufs[i % 2],
            out_hbm.at[pl.ds(out_base + i * _PCHUNK, _PCHUNK)], sem_wr)
    writes[(n_chunks - 1) % 2].wait()
    writes[n_chunks % 2].wait()


def kernel(input, table):
    b, l = input.shape
    n = b * l
    npair = n // 2
    nw = _NC * _NS
    pairs_per_tile = npair // nw
    v, d = table.shape

    flat2 = input.reshape(npair, 2).astype(jnp.int32)
    ue = (2 * flat2[:, 0]).reshape(npair)
    uo = (2 * flat2[:, 1] + 1).reshape(npair)
    tpad = jnp.zeros((_VOCAB_PAD, _DIM), jnp.float32).at[:v].set(table)

    mesh = plsc.VectorSubcoreMesh(core_axis_name="c", subcore_axis_name="s")
    run = pl.kernel(
        _body,
        out_type=jax.ShapeDtypeStruct((npair, _ROW), jnp.float32),
        mesh=mesh,
        scratch_types=[
            pltpu.VMEM_SHARED((2 * _VOCAB_PAD, _ROW), jnp.float32),
            pltpu.VMEM((_VOCAB_PAD // _NS, _DIM), jnp.float32),
            pltpu.VMEM((2 * _VOCAB_PAD // _NS, _ROW), jnp.float32),
            pltpu.VMEM((pairs_per_tile,), jnp.int32),
            pltpu.VMEM((pairs_per_tile,), jnp.int32),
            pltpu.VMEM((_PCHUNK, _ROW), jnp.float32),
            pltpu.VMEM((_PCHUNK, _ROW), jnp.float32),
            pltpu.SemaphoreType.DMA,
            pltpu.SemaphoreType.DMA,
            pltpu.SemaphoreType.DMA,
            pltpu.SemaphoreType.DMA,
        ],
    )
    out = run(ue, uo, tpad)
    return out.reshape(b, l, _DIM)
